# baseline (speedup 1.0000x reference)
import functools

import jax
import jax.numpy as jnp
from jax.experimental import pallas as pl
from jax.experimental.pallas import tpu as pltpu

N_ROWS = 16384
N_COLS = 1000
CHUNK = 512
NCHUNK = N_ROWS // CHUNK
NBUF = 4
NUM_SAVED = N_ROWS // 2


def _row_losses(x, tgt):
    mx = jnp.max(x, axis=1, keepdims=True)
    s = jnp.sum(jnp.exp(x - mx), axis=1)
    lse = mx[:, 0] + jnp.log(s)
    cols = jax.lax.broadcasted_iota(jnp.int32, (CHUNK, N_COLS), 1)
    xt = jnp.sum(jnp.where(cols == tgt[:, None], x, 0.0), axis=1)
    return lse - xt


def _loss_topk_kernel(x_hbm, tgt_ref, out_ref, buf_ref, loss_ref, sems):
    def copy(c):
        return pltpu.make_async_copy(
            x_hbm.at[pl.ds(c * CHUNK, CHUNK), :],
            buf_ref.at[c % NBUF],
            sems.at[c % NBUF],
        )

    for c in range(NBUF):
        copy(c).start()

    for c in range(NCHUNK):
        copy(c).wait()
        x = buf_ref[c % NBUF]
        tgt = tgt_ref[c, 0, :]
        loss_ref[c, 0, :] = _row_losses(x, tgt)
        if c + NBUF < NCHUNK:
            copy(c + NBUF).start()

    loss = loss_ref[...]
    b = jax.lax.bitcast_convert_type(loss, jnp.int32)
    m = jnp.where(b >= 0, b, b ^ jnp.int32(0x7FFFFFFF))
    u = jax.lax.bitcast_convert_type(m, jnp.uint32) ^ jnp.uint32(0x80000000)

    k = jnp.int32(NUM_SAVED)

    def bit_step(bit, acc):
        cand = acc | (jnp.uint32(1) << jnp.uint32(31 - bit))
        cnt = jnp.sum((u >= cand).astype(jnp.int32))
        return jnp.where(cnt >= k, cand, acc)

    sel = jax.lax.fori_loop(0, 32, bit_step, jnp.uint32(0))

    above = u > sel
    c_above = jnp.sum(above.astype(jnp.float32))
    s_above = jnp.sum(jnp.where(above, loss, 0.0))
    mv = jax.lax.bitcast_convert_type(sel ^ jnp.uint32(0x80000000), jnp.int32)
    bv = jnp.where(mv >= 0, mv, mv ^ jnp.int32(0x7FFFFFFF))
    v = jax.lax.bitcast_convert_type(bv, jnp.float32)

    total = s_above + (jnp.float32(NUM_SAVED) - c_above) * v
    out_ref[...] = jnp.reshape(total / jnp.float32(NUM_SAVED), (1, 1))


@jax.jit
def kernel(logits, target):
    tgt = target.astype(jnp.int32).reshape(NCHUNK, 1, CHUNK)
    out = pl.pallas_call(
        _loss_topk_kernel,
        in_specs=[
            pl.BlockSpec(memory_space=pl.ANY),
            pl.BlockSpec(memory_space=pltpu.VMEM),
        ],
        out_specs=pl.BlockSpec(memory_space=pltpu.VMEM),
        out_shape=jax.ShapeDtypeStruct((1, 1), jnp.float32),
        scratch_shapes=[
            pltpu.VMEM((NBUF, CHUNK, N_COLS), jnp.float32),
            pltpu.VMEM((NCHUNK, 1, CHUNK), jnp.float32),
            pltpu.SemaphoreType.DMA((NBUF,)),
        ],
    )(logits, tgt)
    return out[0, 0]
